# Initial kernel scaffold; baseline (speedup 1.0000x reference)
import functools

import jax
import jax.numpy as jnp
from jax import lax
from jax.experimental import pallas as pl
from jax.experimental.pallas import tpu as pltpu
from jax.experimental.pallas import tpu_sc as plsc

_L = 16
_CHUNK = 128


def _rsqrt_nr(x):
    i = lax.bitcast_convert_type(x, jnp.int32)
    i = jnp.int32(0x5F3759DF) - lax.shift_right_logical(i, 1)
    y = lax.bitcast_convert_type(i, jnp.float32)
    for _ in range(3):
        y = y * (jnp.float32(1.5) - jnp.float32(0.5) * x * y * y)
    return y


@functools.cache
def _make_sc_kernel(B, D, NC, NS):
    NW = NC * NS
    bpw = B // NW
    nchunks = bpw // _CHUNK
    nj = D // _L
    mesh = plsc.VectorSubcoreMesh(core_axis_name="c", subcore_axis_name="s")

    @functools.partial(
        pl.kernel,
        mesh=mesh,
        out_type=jax.ShapeDtypeStruct((B,), jnp.float32),
        scratch_types=[
            pltpu.VMEM((nchunks, _CHUNK), jnp.int32),
            pltpu.VMEM((bpw, D), jnp.float32),
            pltpu.VMEM((D,), jnp.float32),
            pltpu.VMEM((bpw,), jnp.float32),
            pltpu.VMEM((bpw,), jnp.float32),
            pltpu.VMEM((bpw,), jnp.float32),
            pltpu.SemaphoreType.DMA,
        ],
    )
    def k(p_hbm, idx_hbm, v_hbm, out_hbm, idx_v, rows_v, v_v, sv_v, ss_v,
          out_v, sem):
        wid = lax.axis_index("s") * NC + lax.axis_index("c")
        base = wid * bpw

        pltpu.sync_copy(v_hbm, v_v)
        for c in range(nchunks):
            pltpu.sync_copy(idx_hbm.at[pl.ds(base + _CHUNK * c, _CHUNK)],
                            idx_v.at[c])
        copies = [
            pltpu.async_copy(p_hbm.at[idx_v.at[c]],
                             rows_v.at[pl.ds(_CHUNK * c, _CHUNK)], sem)
            for c in range(nchunks)
        ]
        for cp in copies:
            cp.wait()

        vjs = [v_v[pl.ds(_L * j, _L)] for j in range(nj)]

        def body(r, carry):
            acc_v = jnp.zeros((_L,), jnp.float32)
            acc_s = jnp.zeros((_L,), jnp.float32)
            for j in range(nj):
                xj = rows_v[r, pl.ds(_L * j, _L)]
                acc_v = acc_v + xj * vjs[j]
                acc_s = acc_s + xj * xj
            sv_v[r] = jnp.sum(acc_v)
            ss_v[r] = jnp.sum(acc_s)
            return carry

        lax.fori_loop(0, bpw, body, 0)

        for m in range(bpw // _L):
            s = pl.ds(_L * m, _L)
            nsq = jnp.maximum(ss_v[s], jnp.float32(1e-24))
            out_v[s] = sv_v[s] * _rsqrt_nr(nsq)
        pltpu.sync_copy(out_v, out_hbm.at[pl.ds(base, bpw)])

    return k


def _proj_body(p_ref, w_ref, c_ref, o_ref):
    o_ref[...] = lax.dot_general(
        p_ref[...], w_ref[...],
        dimension_numbers=(((1,), (1,)), ((), ())),
        preferred_element_type=jnp.float32) * c_ref[...]


def _proj(prompt_embed, text_proj_w, classifier_w):
    out = pl.pallas_call(
        _proj_body,
        out_shape=jax.ShapeDtypeStruct((1, text_proj_w.shape[0]),
                                       jnp.float32),
    )(prompt_embed.reshape(1, -1), text_proj_w, classifier_w)
    return out.reshape(-1)


def kernel(model_id, prompt_embed, P, text_proj_w, classifier_w):
    v = _proj(prompt_embed, text_proj_w, classifier_w)
    info = plsc.get_sparse_core_info()
    sc = _make_sc_kernel(model_id.shape[0], P.shape[1], info.num_cores,
                         info.num_subcores)
    return sc(P, model_id.astype(jnp.int32), v)


# speedup vs baseline: 1.2289x; 1.2289x over previous
import functools

import jax
import jax.numpy as jnp
from jax import lax
from jax.experimental import pallas as pl
from jax.experimental.pallas import tpu as pltpu
from jax.experimental.pallas import tpu_sc as plsc

_L = 16
_CHUNK = 128


def _rsqrt_nr(x):
    i = lax.bitcast_convert_type(x, jnp.int32)
    i = jnp.int32(0x5F3759DF) - lax.shift_right_logical(i, 1)
    y = lax.bitcast_convert_type(i, jnp.float32)
    for _ in range(3):
        y = y * (jnp.float32(1.5) - jnp.float32(0.5) * x * y * y)
    return y


@functools.cache
def _make_sc_kernel(B, D, NC, NS):
    NW = NC * NS
    bpw = B // NW
    nchunks = bpw // _CHUNK
    nj = D // _L
    mesh = plsc.VectorSubcoreMesh(core_axis_name="c", subcore_axis_name="s")

    @functools.partial(
        pl.kernel,
        mesh=mesh,
        compiler_params=pltpu.CompilerParams(needs_layout_passes=False),
        out_type=jax.ShapeDtypeStruct((B,), jnp.float32),
        scratch_types=[
            pltpu.VMEM((nchunks, _CHUNK), jnp.int32),
            pltpu.VMEM((bpw, D), jnp.float32),
            pltpu.VMEM((D,), jnp.float32),
            pltpu.VMEM((_L * _L,), jnp.float32),
            pltpu.VMEM((_L * _L,), jnp.float32),
            pltpu.VMEM((bpw,), jnp.float32),
            pltpu.SemaphoreType.DMA,
        ],
    )
    def k(p_hbm, idx_hbm, v_hbm, out_hbm, idx_v, rows_v, v_v, tv_v, ts_v,
          out_v, sem):
        wid = lax.axis_index("s") * NC + lax.axis_index("c")
        base = wid * bpw

        pltpu.sync_copy(v_hbm, v_v)
        for c in range(nchunks):
            pltpu.sync_copy(idx_hbm.at[pl.ds(base + _CHUNK * c, _CHUNK)],
                            idx_v.at[c])
        copies = [
            pltpu.async_copy(p_hbm.at[idx_v.at[c]],
                             rows_v.at[pl.ds(_CHUNK * c, _CHUNK)], sem)
            for c in range(nchunks)
        ]
        for cp in copies:
            cp.wait()

        vjs = [v_v[pl.ds(_L * j, _L)] for j in range(nj)]
        iot = lax.iota(jnp.int32, _L)

        def body(g, carry):
            rbase = g * _L
            for l in range(_L):
                acc_v = jnp.zeros((_L,), jnp.float32)
                acc_s = jnp.zeros((_L,), jnp.float32)
                for j in range(nj):
                    xj = rows_v[rbase + l, pl.ds(_L * j, _L)]
                    acc_v = acc_v + xj * vjs[j]
                    acc_s = acc_s + xj * xj
                tv_v[pl.ds(_L * l, _L)] = acc_v
                ts_v[pl.ds(_L * l, _L)] = acc_s
            sv = jnp.zeros((_L,), jnp.float32)
            ss = jnp.zeros((_L,), jnp.float32)
            rowsel = iot * _L
            for j in range(_L):
                cj = rowsel + j
                sv = sv + plsc.load_gather(tv_v, [cj])
                ss = ss + plsc.load_gather(ts_v, [cj])
            nsq = jnp.maximum(ss, jnp.float32(1e-24))
            out_v[pl.ds(rbase, _L)] = sv * _rsqrt_nr(nsq)
            return carry

        lax.fori_loop(0, bpw // _L, body, 0)
        pltpu.sync_copy(out_v, out_hbm.at[pl.ds(base, bpw)])

    return k


def _proj_body(p_ref, w_ref, c_ref, o_ref):
    o_ref[...] = lax.dot_general(
        p_ref[...], w_ref[...],
        dimension_numbers=(((1,), (1,)), ((), ())),
        preferred_element_type=jnp.float32) * c_ref[...]


def _proj(prompt_embed, text_proj_w, classifier_w):
    out = pl.pallas_call(
        _proj_body,
        out_shape=jax.ShapeDtypeStruct((1, text_proj_w.shape[0]),
                                       jnp.float32),
    )(prompt_embed.reshape(1, -1), text_proj_w, classifier_w)
    return out.reshape(-1)


def kernel(model_id, prompt_embed, P, text_proj_w, classifier_w):
    v = _proj(prompt_embed, text_proj_w, classifier_w)
    info = plsc.get_sparse_core_info()
    sc = _make_sc_kernel(model_id.shape[0], P.shape[1], info.num_cores,
                         info.num_subcores)
    return sc(P, model_id.astype(jnp.int32), v)
